# Initial kernel scaffold; baseline (speedup 1.0000x reference)
import jax
import jax.numpy as jnp
from jax.experimental import pallas as pl


def kernel(input_ids, weight):
    raise NotImplementedError("write your pallas kernel here")


# speedup vs baseline: 6.3376x; 6.3376x over previous
import functools

import jax
import jax.numpy as jnp
from jax import lax
from jax.experimental import pallas as pl
from jax.experimental.pallas import tpu as pltpu, tpu_sc as plsc

_CHUNK = 128


@functools.partial(jax.jit, static_argnums=(2, 3, 4))
def _sc_gather(ids3, weight, nw, chunks, dim):
    nc = 2
    mesh = plsc.VectorSubcoreMesh(core_axis_name="c", subcore_axis_name="s")
    n_rows = nw * chunks * _CHUNK

    @functools.partial(
        pl.kernel,
        out_type=jax.ShapeDtypeStruct((n_rows, dim), jnp.float32),
        mesh=mesh,
        scratch_types=[
            pltpu.VMEM((chunks, _CHUNK), jnp.int32),
            pltpu.VMEM((_CHUNK, dim), jnp.float32),
            pltpu.SemaphoreType.DMA,
        ],
    )
    def k(ids_hbm, table_hbm, out_hbm, idx_v, rows_v, sem):
        wid = lax.axis_index("s") * nc + lax.axis_index("c")
        base = wid * (chunks * _CHUNK)
        pltpu.sync_copy(ids_hbm.at[wid], idx_v)

        def body(j, carry):
            pltpu.async_copy(table_hbm.at[idx_v.at[j]], rows_v, sem).wait()
            pltpu.sync_copy(rows_v, out_hbm.at[pl.ds(base + j * _CHUNK, _CHUNK)])
            return carry

        lax.fori_loop(0, chunks, body, 0)

    return k(ids3, weight)


def kernel(input_ids, weight):
    b, t = input_ids.shape
    n, dim = weight.shape
    total = b * t
    nw = 32
    assert total % (nw * _CHUNK) == 0
    chunks = total // (nw * _CHUNK)
    ids3 = input_ids.astype(jnp.int32).reshape(nw, chunks, _CHUNK)
    out = _sc_gather(ids3, weight, nw, chunks, dim)
    return out.reshape(b, t, dim)


# speedup vs baseline: 9.2229x; 1.4553x over previous
import functools

import jax
import jax.numpy as jnp
from jax import lax
from jax.experimental import pallas as pl
from jax.experimental.pallas import tpu as pltpu, tpu_sc as plsc

_CHUNK = 128
_NBUF = 4


@functools.partial(jax.jit, static_argnums=(2, 3, 4))
def _sc_gather(ids3, weight, nw, chunks, dim):
    nc = 2
    mesh = plsc.VectorSubcoreMesh(core_axis_name="c", subcore_axis_name="s")
    n_rows = nw * chunks * _CHUNK
    ngroups = chunks // _NBUF
    assert chunks % _NBUF == 0 and ngroups >= 2

    @functools.partial(
        pl.kernel,
        out_type=jax.ShapeDtypeStruct((n_rows, dim), jnp.float32),
        mesh=mesh,
        scratch_types=[
            pltpu.VMEM((chunks, _CHUNK), jnp.int32),
            pltpu.VMEM((_NBUF, _CHUNK, dim), jnp.float32),
            pltpu.SemaphoreType.DMA((_NBUF,)),
            pltpu.SemaphoreType.DMA((_NBUF,)),
        ],
    )
    def k(ids_hbm, table_hbm, out_hbm, idx_v, rows_v, gsem, osem):
        wid = lax.axis_index("s") * nc + lax.axis_index("c")
        base = wid * (chunks * _CHUNK)
        pltpu.sync_copy(ids_hbm.at[wid], idx_v)

        def gather(j, b):
            return pltpu.make_async_copy(
                table_hbm.at[idx_v.at[j]], rows_v.at[b], gsem.at[b])

        def writeout(j, b):
            return pltpu.make_async_copy(
                rows_v.at[b], out_hbm.at[pl.ds(base + j * _CHUNK, _CHUNK)],
                osem.at[b])

        for b in range(_NBUF):
            gather(b, b).start()

        def group(g, carry):
            j0 = g * _NBUF
            for b in range(_NBUF):
                j = j0 + b
                gather(j, b).wait()
                writeout(j, b).start()
                writeout(j, b).wait()
                gather(j + _NBUF, b).start()
            return carry

        lax.fori_loop(0, ngroups - 1, group, 0)

        j0 = (ngroups - 1) * _NBUF
        for b in range(_NBUF):
            gather(j0 + b, b).wait()
            writeout(j0 + b, b).start()
        for b in range(_NBUF):
            writeout(j0 + b, b).wait()

    return k(ids3, weight)


def kernel(input_ids, weight):
    b, t = input_ids.shape
    n, dim = weight.shape
    total = b * t
    nw = 32
    assert total % (nw * _CHUNK) == 0
    chunks = total // (nw * _CHUNK)
    ids3 = input_ids.astype(jnp.int32).reshape(nw, chunks, _CHUNK)
    out = _sc_gather(ids3, weight, nw, chunks, dim)
    return out.reshape(b, t, dim)
